# Initial kernel scaffold; baseline (speedup 1.0000x reference)
import jax
import jax.numpy as jnp
from jax.experimental import pallas as pl


def kernel(sign_table, stroke_table, sign_ids, stroke_ids):
    raise NotImplementedError("write your pallas kernel here")


# speedup vs baseline: 6.4482x; 6.4482x over previous
import functools

import jax
import jax.numpy as jnp
from jax import lax
from jax.experimental import pallas as pl
from jax.experimental.pallas import tpu as pltpu
from jax.experimental.pallas import tpu_sc as plsc

_N = 4096 * 200
_D = 32
_NW = 32
_PER_W = _N // _NW
_CH = 1600
_NCH = _PER_W // _CH

_mesh = plsc.VectorSubcoreMesh(core_axis_name="c", subcore_axis_name="s")


@functools.partial(
    pl.kernel,
    mesh=_mesh,
    compiler_params=pltpu.CompilerParams(use_tc_tiling_on_sc=False),
    out_type=jax.ShapeDtypeStruct((_N, 2 * _D), jnp.float32),
    scratch_types=[
        pltpu.VMEM((_CH,), jnp.int32),
        pltpu.VMEM((_CH,), jnp.int32),
        pltpu.VMEM((_CH, _D), jnp.float32),
        pltpu.VMEM((_CH, _D), jnp.float32),
        pltpu.SemaphoreType.DMA,
        pltpu.SemaphoreType.DMA,
    ],
)
def _embed_gather(sign_hbm, stroke_hbm, sid_hbm, tid_hbm, out_hbm,
                  sidx_v, tidx_v, srows_v, trows_v, sem_s, sem_t):
    wid = lax.axis_index("s") * 2 + lax.axis_index("c")

    def body(c, _):
        base = wid * _PER_W + c * _CH
        pltpu.sync_copy(sid_hbm.at[pl.ds(base, _CH)], sidx_v)
        pltpu.sync_copy(tid_hbm.at[pl.ds(base, _CH)], tidx_v)
        g_s = pltpu.async_copy(sign_hbm.at[sidx_v], srows_v, sem_s)
        g_t = pltpu.async_copy(stroke_hbm.at[tidx_v], trows_v, sem_t)
        g_s.wait()
        g_t.wait()
        pltpu.sync_copy(srows_v, out_hbm.at[pl.ds(base, _CH), pl.ds(0, _D)])
        pltpu.sync_copy(trows_v, out_hbm.at[pl.ds(base, _CH), pl.ds(_D, _D)])
        return 0

    lax.fori_loop(0, _NCH, body, 0)


def kernel(sign_table, stroke_table, sign_ids, stroke_ids):
    batch, seq = sign_ids.shape
    out = _embed_gather(sign_table, stroke_table,
                        sign_ids.reshape(-1), stroke_ids.reshape(-1))
    return out.reshape(batch, seq, 2 * _D)


# speedup vs baseline: 6.4560x; 1.0012x over previous
import functools

import jax
import jax.numpy as jnp
from jax import lax
from jax.experimental import pallas as pl
from jax.experimental.pallas import tpu as pltpu
from jax.experimental.pallas import tpu_sc as plsc

_N = 4096 * 200
_D = 32
_NW = 32
_PER_W = _N // _NW
_CH = 800
_NCH = _PER_W // _CH

_mesh = plsc.VectorSubcoreMesh(core_axis_name="c", subcore_axis_name="s")


@functools.partial(
    pl.kernel,
    mesh=_mesh,
    compiler_params=pltpu.CompilerParams(use_tc_tiling_on_sc=False),
    out_type=jax.ShapeDtypeStruct((_N, 2 * _D), jnp.float32),
    scratch_types=[
        pltpu.VMEM((2, _CH), jnp.int32),
        pltpu.VMEM((2, _CH), jnp.int32),
        pltpu.VMEM((2, _CH, _D), jnp.float32),
        pltpu.VMEM((2, _CH, _D), jnp.float32),
        pltpu.SemaphoreType.DMA,
        pltpu.SemaphoreType.DMA,
        pltpu.SemaphoreType.DMA,
        pltpu.SemaphoreType.DMA,
        pltpu.SemaphoreType.DMA,
        pltpu.SemaphoreType.DMA,
    ],
)
def _embed_gather(sign_hbm, stroke_hbm, sid_hbm, tid_hbm, out_hbm,
                  sidx_v, tidx_v, srows_v, trows_v,
                  isem0, isem1, gsem0, gsem1, ssem0, ssem1):
    wid = lax.axis_index("s") * 2 + lax.axis_index("c")
    base0 = wid * _PER_W
    isem = (isem0, isem1)
    gsem = (gsem0, gsem1)
    ssem = (ssem0, ssem1)

    pend_idx = [None, None]
    pend_gather = [None, None]
    pend_scatter = [None, None]

    def prefetch_idx(c):
        b = c & 1
        base = base0 + c * _CH
        i1 = pltpu.async_copy(sid_hbm.at[pl.ds(base, _CH)], sidx_v.at[b], isem[b])
        i2 = pltpu.async_copy(tid_hbm.at[pl.ds(base, _CH)], tidx_v.at[b], isem[b])
        pend_idx[b] = (i1, i2)

    def drain_scatter(b):
        if pend_scatter[b] is not None:
            for d in pend_scatter[b]:
                d.wait()
            pend_scatter[b] = None

    def issue_scatter(b):
        if pend_gather[b] is not None:
            descs, base = pend_gather[b]
            for d in descs:
                d.wait()
            s1 = pltpu.async_copy(
                srows_v.at[b], out_hbm.at[pl.ds(base, _CH), pl.ds(0, _D)], ssem[b])
            s2 = pltpu.async_copy(
                trows_v.at[b], out_hbm.at[pl.ds(base, _CH), pl.ds(_D, _D)], ssem[b])
            pend_scatter[b] = (s1, s2)
            pend_gather[b] = None

    prefetch_idx(0)
    for c in range(_NCH):
        b = c & 1
        base = base0 + c * _CH
        drain_scatter(b)
        for d in pend_idx[b]:
            d.wait()
        pend_idx[b] = None
        g1 = pltpu.async_copy(sign_hbm.at[sidx_v.at[b]], srows_v.at[b], gsem[b])
        g2 = pltpu.async_copy(stroke_hbm.at[tidx_v.at[b]], trows_v.at[b], gsem[b])
        pend_gather[b] = ((g1, g2), base)
        issue_scatter(1 - b)
        if c + 1 < _NCH:
            prefetch_idx(c + 1)
    issue_scatter((_NCH - 1) & 1)
    drain_scatter(0)
    drain_scatter(1)


def kernel(sign_table, stroke_table, sign_ids, stroke_ids):
    batch, seq = sign_ids.shape
    out = _embed_gather(sign_table, stroke_table,
                        sign_ids.reshape(-1), stroke_ids.reshape(-1))
    return out.reshape(batch, seq, 2 * _D)
